# baseline (speedup 1.0000x reference)
import jax
import jax.numpy as jnp
from jax.experimental import pallas as pl
from jax.experimental.pallas import tpu as pltpu

N_TOK = 8192
D_MODEL = 2048
BLOCK_M = 2048


def _matvec_bias_kernel(x_ref, w_ref, b_ref, o_ref):
    x = x_ref[:].reshape(BLOCK_M, D_MODEL)
    o_ref[:] = (
        jnp.dot(x, w_ref[:], preferred_element_type=jnp.float32) + b_ref[0, 0]
    )


def kernel(X, W_g, b_g, W_e, b_e):
    B = X.shape[0]
    D = X.shape[-1]
    b2d = b_e.reshape(1, 1)
    grid = (B // BLOCK_M,)
    return pl.pallas_call(
        _matvec_bias_kernel,
        grid=grid,
        in_specs=[
            pl.BlockSpec((BLOCK_M, 1, D), lambda i: (i, 0, 0)),
            pl.BlockSpec((D, 1), lambda i: (0, 0)),
            pl.BlockSpec((1, 1), lambda i: (0, 0)),
        ],
        out_specs=pl.BlockSpec((BLOCK_M, 1), lambda i: (i, 0)),
        out_shape=jax.ShapeDtypeStruct((B, 1), X.dtype),
        compiler_params=pltpu.CompilerParams(
            dimension_semantics=("parallel",),
        ),
    )(X, W_e, b2d)


# speedup vs baseline: 1.0111x; 1.0111x over previous
import jax
import jax.numpy as jnp
from jax.experimental import pallas as pl
from jax.experimental.pallas import tpu as pltpu

N_TOK = 8192
D_MODEL = 2048
BLOCK_M = 1024


def _matvec_bias_kernel(x_ref, w_ref, b_ref, o_ref):
    x = x_ref[:].reshape(BLOCK_M, D_MODEL)
    o_ref[:] = (
        jnp.dot(x, w_ref[:], preferred_element_type=jnp.float32) + b_ref[0, 0]
    )


def kernel(X, W_g, b_g, W_e, b_e):
    B = X.shape[0]
    D = X.shape[-1]
    b2d = b_e.reshape(1, 1)
    grid = (B // BLOCK_M,)
    return pl.pallas_call(
        _matvec_bias_kernel,
        grid=grid,
        in_specs=[
            pl.BlockSpec((BLOCK_M, 1, D), lambda i: (i, 0, 0)),
            pl.BlockSpec((D, 1), lambda i: (0, 0)),
            pl.BlockSpec((1, 1), lambda i: (0, 0)),
        ],
        out_specs=pl.BlockSpec((BLOCK_M, 1), lambda i: (i, 0)),
        out_shape=jax.ShapeDtypeStruct((B, 1), X.dtype),
        compiler_params=pltpu.CompilerParams(
            dimension_semantics=("parallel",),
        ),
    )(X, W_e, b2d)
